# baseline (speedup 1.0000x reference)
import functools

import jax
import jax.numpy as jnp
from jax import lax
from jax.experimental import pallas as pl
from jax.experimental.pallas import tpu as pltpu
from jax.experimental.pallas import tpu_sc as plsc

BATCH = 16384
HALF = BATCH // 2
D = 64
NC = 2
NS = 16
L = 16
NW = NC * NS
BPW = HALF // NW
CHUNK = 128
NCHUNK = BPW // CHUNK

_mesh = plsc.VectorSubcoreMesh(core_axis_name="c", subcore_axis_name="s")


@functools.partial(
    pl.kernel,
    mesh=_mesh,
    out_type=jax.ShapeDtypeStruct((HALF, D), jnp.float32),
    scratch_types=[
        pltpu.VMEM((NCHUNK, CHUNK), jnp.int32),
        pltpu.VMEM((CHUNK, 2 * D), jnp.float32),
        pltpu.VMEM((CHUNK, 2 * D), jnp.float32),
        pltpu.VMEM((CHUNK, D), jnp.float32),
        pltpu.VMEM((CHUNK, D), jnp.float32),
        pltpu.VMEM((CHUNK, D), jnp.float32),
        pltpu.VMEM((CHUNK, D), jnp.float32),
        pltpu.SemaphoreType.DMA,
        pltpu.SemaphoreType.DMA,
        pltpu.SemaphoreType.DMA,
        pltpu.SemaphoreType.DMA,
        pltpu.SemaphoreType.DMA,
        pltpu.SemaphoreType.DMA,
    ],
)
def _film_half(x_hbm, idx_hbm, film_hbm, out_hbm,
               idx_v, rows0, rows1, xb0, xb1, ob0, ob1,
               gs0, gs1, xs0, xs1, os0, os1):
    rows = (rows0, rows1)
    xb = (xb0, xb1)
    ob = (ob0, ob1)
    gsem = (gs0, gs1)
    xsem = (xs0, xs1)
    osem = (os0, os1)

    wid = lax.axis_index("s") * NC + lax.axis_index("c")
    base = wid * BPW
    pltpu.sync_copy(idx_hbm.at[wid], idx_v)

    gathers = [None, None]
    xcopies = [None, None]
    ostores = [None, None]

    def start(c):
        b = c % 2
        gathers[b] = pltpu.async_copy(film_hbm.at[idx_v.at[c]], rows[b], gsem[b])
        xcopies[b] = pltpu.async_copy(
            x_hbm.at[pl.ds(base + c * CHUNK, CHUNK)], xb[b], xsem[b])

    start(0)
    for c in range(NCHUNK):
        b = c % 2
        if c + 1 < NCHUNK:
            start(c + 1)
        gathers[b].wait()
        xcopies[b].wait()
        if c >= 2:
            ostores[b].wait()

        @plsc.parallel_loop(0, CHUNK, unroll=8)
        def body(r):
            for j in range(D // L):
                sl = pl.ds(j * L, L)
                ob[b][r, sl] = rows[b][r, sl] * xb[b][r, sl] \
                    + rows[b][r, pl.ds(D + j * L, L)]

        ostores[b] = pltpu.async_copy(
            ob[b], out_hbm.at[pl.ds(base + c * CHUNK, CHUNK)], osem[b])

    for b in range(min(2, NCHUNK)):
        ostores[b].wait()


def kernel(x, cell_line, film):
    idx = cell_line.reshape(2, NW, NCHUNK, CHUNK)
    out_a = _film_half(x[:HALF], idx[0], film)
    out_b = _film_half(x[HALF:], idx[1], film)
    out = jnp.concatenate([out_a, out_b], axis=0)
    return (out, cell_line)


# speedup vs baseline: 1.2684x; 1.2684x over previous
import functools

import jax
import jax.numpy as jnp
from jax import lax
from jax.experimental import pallas as pl
from jax.experimental.pallas import tpu as pltpu
from jax.experimental.pallas import tpu_sc as plsc

BATCH = 16384
D = 64
NC = 2
NS = 16
L = 16
NW = NC * NS
BPW = BATCH // NW
CHUNK = 128
NCHUNK = BPW // CHUNK

_mesh = plsc.VectorSubcoreMesh(core_axis_name="c", subcore_axis_name="s")


@functools.partial(
    pl.kernel,
    mesh=_mesh,
    out_type=jax.ShapeDtypeStruct((BATCH, D), jnp.float32),
    scratch_types=[
        pltpu.VMEM((NCHUNK, CHUNK), jnp.int32),
        pltpu.VMEM((CHUNK, 2 * D), jnp.float32),
        pltpu.VMEM((CHUNK, 2 * D), jnp.float32),
        pltpu.VMEM((CHUNK, D), jnp.float32),
        pltpu.VMEM((CHUNK, D), jnp.float32),
        pltpu.VMEM((CHUNK, D), jnp.float32),
        pltpu.VMEM((CHUNK, D), jnp.float32),
        pltpu.SemaphoreType.DMA,
        pltpu.SemaphoreType.DMA,
        pltpu.SemaphoreType.DMA,
        pltpu.SemaphoreType.DMA,
        pltpu.SemaphoreType.DMA,
        pltpu.SemaphoreType.DMA,
    ],
)
def _film(x_hbm, idx_hbm, film_hbm, out_hbm,
          idx_v, rows0, rows1, xb0, xb1, ob0, ob1,
          gs0, gs1, xs0, xs1, os0, os1):
    rows = (rows0, rows1)
    xb = (xb0, xb1)
    ob = (ob0, ob1)
    gsem = (gs0, gs1)
    xsem = (xs0, xs1)
    osem = (os0, os1)

    wid = lax.axis_index("s") * NC + lax.axis_index("c")
    base = wid * BPW
    pltpu.sync_copy(idx_hbm.at[wid], idx_v)

    gathers = [None, None]
    xcopies = [None, None]
    ostores = [None, None]

    def start(c):
        b = c % 2
        gathers[b] = pltpu.async_copy(film_hbm.at[idx_v.at[c]], rows[b], gsem[b])
        xcopies[b] = pltpu.async_copy(
            x_hbm.at[pl.ds(base + c * CHUNK, CHUNK)], xb[b], xsem[b])

    start(0)
    for c in range(NCHUNK):
        b = c % 2
        if c + 1 < NCHUNK:
            nb = (c + 1) % 2
            start(c + 1)
        gathers[b].wait()
        xcopies[b].wait()
        if c >= 2:
            ostores[b].wait()

        @plsc.parallel_loop(0, CHUNK, unroll=8)
        def body(r):
            for j in range(D // L):
                sl = pl.ds(j * L, L)
                ob[b][r, sl] = rows[b][r, sl] * xb[b][r, sl] \
                    + rows[b][r, pl.ds(D + j * L, L)]

        ostores[b] = pltpu.async_copy(
            ob[b], out_hbm.at[pl.ds(base + c * CHUNK, CHUNK)], osem[b])

    ostores[(NCHUNK - 2) % 2].wait()
    ostores[(NCHUNK - 1) % 2].wait()


def kernel(x, cell_line, film):
    idx = cell_line.reshape(NW, NCHUNK, CHUNK)
    out = _film(x, idx, film)
    return (out, cell_line)
